# Initial kernel scaffold; baseline (speedup 1.0000x reference)
import jax
import jax.numpy as jnp
from jax.experimental import pallas as pl


def kernel(logits, targets, W_head, P0, S0, P1, S1, P2, S2):
    raise NotImplementedError("write your pallas kernel here")


# speedup vs baseline: 2.1444x; 2.1444x over previous
import jax
import jax.numpy as jnp
from jax.experimental import pallas as pl
from jax.experimental.pallas import tpu as pltpu

N = 8192
C = 1024
B = 256
C0 = 4000
NTAIL = 3
HDIM = C0 + NTAIL
HPAD = 4096
KC = 2048
CUT = (4000, 20000, 60000, 100000)
TD = (256, 64, 16)
TOFF = (0, 256, 384)
TW = 512


def _nchunks(j):
    return (CUT[j + 1] - CUT[j] + KC - 1) // KC


def _k1_body(x_ref, wh_ref, p0_ref, p1_ref, p2_ref, tgt_ref,
             t_ref, hq_ref, wbf_ref):
    b = pl.program_id(0)

    @pl.when(b == 0)
    def _():
        wbf_ref[...] = wh_ref[...].astype(jnp.bfloat16)

    x = x_ref[...].astype(jnp.bfloat16)
    u = jnp.dot(x, wbf_ref[...], preferred_element_type=jnp.float32)
    col = jax.lax.broadcasted_iota(jnp.int32, (B, HPAD), 1)
    tgt = tgt_ref[0, 0, :]
    hidx = jnp.clip(tgt, 0, C0 - 1)
    eu = jnp.where(col < HDIM, jnp.exp(u), 0.0)
    s = jnp.sum(eu, axis=1)
    tval = jnp.sum(jnp.where(col == hidx[:, None], u, 0.0), axis=1)
    cl0 = jnp.sum(jnp.where(col == C0, u, 0.0), axis=1)
    cl1 = jnp.sum(jnp.where(col == C0 + 1, u, 0.0), axis=1)
    cl2 = jnp.sum(jnp.where(col == C0 + 2, u, 0.0), axis=1)
    hq_ref[0, 0, :] = jnp.log(s)
    hq_ref[0, 1, :] = tval
    hq_ref[0, 2, :] = cl0
    hq_ref[0, 3, :] = cl1
    hq_ref[0, 4, :] = cl2
    zero = jnp.zeros((B,), jnp.float32)
    hq_ref[0, 5, :] = zero
    hq_ref[0, 6, :] = zero
    hq_ref[0, 7, :] = zero

    t0 = jnp.dot(x, p0_ref[...].astype(jnp.bfloat16), preferred_element_type=jnp.float32)
    t1 = jnp.dot(x, p1_ref[...].astype(jnp.bfloat16), preferred_element_type=jnp.float32)
    t2 = jnp.dot(x, p2_ref[...].astype(jnp.bfloat16), preferred_element_type=jnp.float32)
    t_ref[:, TOFF[0]:TOFF[0] + TD[0]] = t0.astype(jnp.bfloat16)
    t_ref[:, TOFF[1]:TOFF[1] + TD[1]] = t1.astype(jnp.bfloat16)
    t_ref[:, TOFF[2]:TOFF[2] + TD[2]] = t2.astype(jnp.bfloat16)


def _k2_body(tgt_ref, hq_ref, t_ref, s0_ref, s1_ref, s2_ref,
             loss_ref, accs_ref, acct_ref):
    i = pl.program_id(0)
    k = pl.program_id(1)
    b = pl.program_id(2)
    nk_last = _nchunks(2)

    tgt = tgt_ref[0, 0, :]

    def tail(j, s_chunk_ref):
        cnt = CUT[j + 1] - CUT[j]

        @pl.when(k < _nchunks(j))
        def _():
            tb = t_ref[pl.ds(b * B, B), TOFF[j]:TOFF[j] + TD[j]]
            sc = s_chunk_ref[...].astype(jnp.bfloat16)
            uu = jnp.dot(tb, sc, preferred_element_type=jnp.float32)
            col = k * KC + jax.lax.broadcasted_iota(jnp.int32, (B, KC), 1)
            rel = jnp.clip(tgt - CUT[j], 0, cnt - 1)
            eu = jnp.where(col < cnt, jnp.exp(uu), 0.0)
            sp = jnp.sum(eu, axis=1)
            tv = jnp.sum(jnp.where(col == rel[:, None], uu, 0.0), axis=1)
            prev_s = jnp.where(k == 0, 0.0, accs_ref[j, b, :])
            prev_t = jnp.where(k == 0, 0.0, acct_ref[j, b, :])
            accs_ref[j, b, :] = prev_s + sp
            acct_ref[j, b, :] = prev_t + tv

    @pl.when(i == 0)
    def _():
        tail(0, s0_ref)

    @pl.when(i == 1)
    def _():
        tail(1, s1_ref)

    @pl.when(i == 2)
    def _():
        tail(2, s2_ref)

    @pl.when((i == 2) & (k == nk_last - 1))
    def _():
        ls_head = hq_ref[0, 0, :]
        lp = hq_ref[0, 1, :] - ls_head
        for j in range(NTAIL):
            lo, hi = CUT[j], CUT[j + 1]
            lp_j = (hq_ref[0, 2 + j, :] - ls_head
                    + acct_ref[j, b, :] - jnp.log(accs_ref[j, b, :]))
            lp = jnp.where((tgt >= lo) & (tgt < hi), lp_j, lp)
        vec = jnp.sum(lp.reshape(B // 128, 128), axis=0) * (-1.0 / N)
        prev = jnp.where(b == 0, 0.0, loss_ref[0, :])
        acc = prev + vec
        total = jnp.sum(acc)
        loss_ref[0, :] = jnp.where(b == (N // B) - 1,
                                   jnp.full((128,), total), acc)


def kernel(logits, targets, W_head, P0, S0, P1, S1, P2, S2):
    whp = jnp.pad(W_head, ((0, 0), (0, HPAD - HDIM)))
    tgt3 = targets.astype(jnp.int32).reshape(N // B, 1, B)

    t_buf, hq = pl.pallas_call(
        _k1_body,
        grid=(N // B,),
        in_specs=[
            pl.BlockSpec((B, C), lambda b: (b, 0)),
            pl.BlockSpec((C, HPAD), lambda b: (0, 0)),
            pl.BlockSpec((C, TD[0]), lambda b: (0, 0)),
            pl.BlockSpec((C, TD[1]), lambda b: (0, 0)),
            pl.BlockSpec((C, TD[2]), lambda b: (0, 0)),
            pl.BlockSpec((1, 1, B), lambda b: (b, 0, 0)),
        ],
        out_specs=[
            pl.BlockSpec((B, TW), lambda b: (b, 0)),
            pl.BlockSpec((1, 8, B), lambda b: (b, 0, 0)),
        ],
        out_shape=[
            jax.ShapeDtypeStruct((N, TW), jnp.bfloat16),
            jax.ShapeDtypeStruct((N // B, 8, B), jnp.float32),
        ],
        scratch_shapes=[pltpu.VMEM((C, HPAD), jnp.bfloat16)],
    )(logits, whp, P0, P1, P2, tgt3)

    nk = _nchunks(2)
    loss = pl.pallas_call(
        _k2_body,
        grid=(NTAIL, nk, N // B),
        in_specs=[
            pl.BlockSpec((1, 1, B), lambda i, k, b: (b, 0, 0)),
            pl.BlockSpec((1, 8, B), lambda i, k, b: (b, 0, 0)),
            pl.BlockSpec((N, TW), lambda i, k, b: (0, 0)),
            pl.BlockSpec((TD[0], KC),
                         lambda i, k, b: (0, jnp.where(i == 0, jnp.minimum(k, _nchunks(0) - 1), 0))),
            pl.BlockSpec((TD[1], KC),
                         lambda i, k, b: (0, jnp.where(i == 1, k, 0))),
            pl.BlockSpec((TD[2], KC),
                         lambda i, k, b: (0, jnp.where(i == 2, k, 0))),
        ],
        out_specs=pl.BlockSpec((1, 128), lambda i, k, b: (0, 0)),
        out_shape=jax.ShapeDtypeStruct((1, 128), jnp.float32),
        scratch_shapes=[
            pltpu.VMEM((NTAIL, N // B, B), jnp.float32),
            pltpu.VMEM((NTAIL, N // B, B), jnp.float32),
        ],
    )(tgt3, hq, t_buf, S0, S1, S2)

    return loss[0, 0]


# speedup vs baseline: 2.5693x; 1.1981x over previous
import jax
import jax.numpy as jnp
from jax.experimental import pallas as pl
from jax.experimental.pallas import tpu as pltpu

N = 8192
C = 1024
B = 256
C0 = 4000
NTAIL = 3
HDIM = C0 + NTAIL
HPAD = 4096
KC = 2048
CUT = (4000, 20000, 60000, 100000)
TD = (256, 64, 16)
TOFF = (0, 256, 384)
TW = 512


def _nchunks(j):
    return (CUT[j + 1] - CUT[j] + KC - 1) // KC


def _k1_body(x_ref, wh_ref, p0_ref, p1_ref, p2_ref, tgt_ref, t_ref, hq_ref):
    x = x_ref[...].astype(jnp.bfloat16)
    u = jnp.dot(x, wh_ref[...], preferred_element_type=jnp.float32)
    lane = jax.lax.broadcasted_iota(jnp.int32, (1, HPAD), 1)
    tgt = tgt_ref[0, 0, :]
    hidx = jnp.clip(tgt, 0, C0 - 1)
    eu = jnp.where(lane < HDIM, jnp.exp(u), 0.0)
    s = jnp.sum(eu, axis=1)
    tval = jnp.sum(jnp.where(lane == hidx[:, None], u, 0.0), axis=1)
    hq_ref[0, 0, :] = jnp.log(s)
    hq_ref[0, 1, :] = tval
    hq_ref[0, 2, :] = u[:, C0]
    hq_ref[0, 3, :] = u[:, C0 + 1]
    hq_ref[0, 4, :] = u[:, C0 + 2]
    zero = jnp.zeros((B,), jnp.float32)
    hq_ref[0, 5, :] = zero
    hq_ref[0, 6, :] = zero
    hq_ref[0, 7, :] = zero

    t0 = jnp.dot(x, p0_ref[...], preferred_element_type=jnp.float32)
    t1 = jnp.dot(x, p1_ref[...], preferred_element_type=jnp.float32)
    t2 = jnp.dot(x, p2_ref[...], preferred_element_type=jnp.float32)
    t_ref[:, TOFF[0]:TOFF[0] + TD[0]] = t0.astype(jnp.bfloat16)
    t_ref[:, TOFF[1]:TOFF[1] + TD[1]] = t1.astype(jnp.bfloat16)
    t_ref[:, TOFF[2]:TOFF[2] + TD[2]] = t2.astype(jnp.bfloat16)


def _k2_body(tgt_ref, hq_ref, t_ref, s0_ref, s1_ref, s2_ref,
             loss_ref, accs_ref, acct_ref):
    i = pl.program_id(0)
    k = pl.program_id(1)
    lane = jax.lax.broadcasted_iota(jnp.int32, (1, KC), 1)

    def tail(j, s_chunk_ref):
        cnt = CUT[j + 1] - CUT[j]

        @pl.when(k < _nchunks(j))
        def _():
            sc = s_chunk_ref[...]
            base = k * KC
            vmask = (base + lane) < cnt

            def body(b, carry):
                tb = t_ref[pl.ds(b * B, B), TOFF[j]:TOFF[j] + TD[j]]
                uu = jnp.dot(tb, sc, preferred_element_type=jnp.float32)
                tgt = tgt_ref[b, :]
                relk = (jnp.clip(tgt - CUT[j], 0, cnt - 1) - base)[:, None]
                eu = jnp.where(vmask, jnp.exp(uu), 0.0)
                sp = jnp.sum(eu, axis=1)
                tv = jnp.sum(jnp.where(lane == relk, uu, 0.0), axis=1)
                prev_s = jnp.where(k == 0, 0.0, accs_ref[j, b, :])
                prev_t = jnp.where(k == 0, 0.0, acct_ref[j, b, :])
                accs_ref[j, b, :] = prev_s + sp
                acct_ref[j, b, :] = prev_t + tv
                return carry

            jax.lax.fori_loop(0, N // B, body, 0)

    @pl.when(i == 0)
    def _():
        tail(0, s0_ref)

    @pl.when(i == 1)
    def _():
        tail(1, s1_ref)

    @pl.when(i == 2)
    def _():
        tail(2, s2_ref)

    @pl.when((i == 2) & (k == _nchunks(2) - 1))
    def _():
        def cbody(b, acc_vec):
            tgt = tgt_ref[b, :]
            ls_head = hq_ref[b, 0, :]
            lp = hq_ref[b, 1, :] - ls_head
            for j in range(NTAIL):
                lo, hi = CUT[j], CUT[j + 1]
                lp_j = (hq_ref[b, 2 + j, :] - ls_head
                        + acct_ref[j, b, :] - jnp.log(accs_ref[j, b, :]))
                lp = jnp.where((tgt >= lo) & (tgt < hi), lp_j, lp)
            return acc_vec + jnp.sum(lp.reshape(B // 128, 128), axis=0)

        acc = jax.lax.fori_loop(0, N // B, cbody,
                                jnp.zeros((128,), jnp.float32))
        total = jnp.sum(acc) * (-1.0 / N)
        loss_ref[0, :] = jnp.full((128,), total)


def kernel(logits, targets, W_head, P0, S0, P1, S1, P2, S2):
    whp = jnp.pad(W_head, ((0, 0), (0, HPAD - HDIM))).astype(jnp.bfloat16)
    p0b = P0.astype(jnp.bfloat16)
    p1b = P1.astype(jnp.bfloat16)
    p2b = P2.astype(jnp.bfloat16)
    s0b = S0.astype(jnp.bfloat16)
    s1b = S1.astype(jnp.bfloat16)
    s2b = S2.astype(jnp.bfloat16)
    tgt2 = targets.astype(jnp.int32).reshape(N // B, B)
    tgt3 = tgt2.reshape(N // B, 1, B)

    t_buf, hq = pl.pallas_call(
        _k1_body,
        grid=(N // B,),
        in_specs=[
            pl.BlockSpec((B, C), lambda b: (b, 0)),
            pl.BlockSpec((C, HPAD), lambda b: (0, 0)),
            pl.BlockSpec((C, TD[0]), lambda b: (0, 0)),
            pl.BlockSpec((C, TD[1]), lambda b: (0, 0)),
            pl.BlockSpec((C, TD[2]), lambda b: (0, 0)),
            pl.BlockSpec((1, 1, B), lambda b: (b, 0, 0)),
        ],
        out_specs=[
            pl.BlockSpec((B, TW), lambda b: (b, 0)),
            pl.BlockSpec((1, 8, B), lambda b: (b, 0, 0)),
        ],
        out_shape=[
            jax.ShapeDtypeStruct((N, TW), jnp.bfloat16),
            jax.ShapeDtypeStruct((N // B, 8, B), jnp.float32),
        ],
    )(logits, whp, p0b, p1b, p2b, tgt3)

    nk = _nchunks(2)
    loss = pl.pallas_call(
        _k2_body,
        grid=(NTAIL, nk),
        in_specs=[
            pl.BlockSpec((N // B, B), lambda i, k: (0, 0)),
            pl.BlockSpec((N // B, 8, B), lambda i, k: (0, 0, 0)),
            pl.BlockSpec((N, TW), lambda i, k: (0, 0)),
            pl.BlockSpec((TD[0], KC),
                         lambda i, k: (0, jnp.where(i == 0, jnp.minimum(k, _nchunks(0) - 1), 0))),
            pl.BlockSpec((TD[1], KC),
                         lambda i, k: (0, jnp.where(i == 1, k, 0))),
            pl.BlockSpec((TD[2], KC),
                         lambda i, k: (0, jnp.where(i == 2, k, 0))),
        ],
        out_specs=pl.BlockSpec((1, 128), lambda i, k: (0, 0)),
        out_shape=jax.ShapeDtypeStruct((1, 128), jnp.float32),
        scratch_shapes=[
            pltpu.VMEM((NTAIL, N // B, B), jnp.float32),
            pltpu.VMEM((NTAIL, N // B, B), jnp.float32),
        ],
    )(tgt2, hq, t_buf, s0b, s1b, s2b)

    return loss[0, 0]


# speedup vs baseline: 3.5316x; 1.3745x over previous
import jax
import jax.numpy as jnp
from jax.experimental import pallas as pl
from jax.experimental.pallas import tpu as pltpu

N = 8192
C = 1024
B = 256
C0 = 4000
NTAIL = 3
HDIM = C0 + NTAIL
HPAD = 4096
KC = 2048
CUT = (4000, 20000, 60000, 100000)
TD = (256, 64, 16)
TOFF = (0, 256, 384)
TW = 512


def _nchunks(j):
    return (CUT[j + 1] - CUT[j] + KC - 1) // KC


def _npad(j):
    return _nchunks(j) * KC - (CUT[j + 1] - CUT[j])


def _fold128(v):
    r, m = v.shape
    acc = v[:, 0:128]
    for c in range(1, m // 128):
        acc = acc + v[:, 128 * c:128 * (c + 1)]
    return acc


def _k1_body(x_ref, wh_ref, p0_ref, p1_ref, p2_ref, tgt_ref, t_ref, hq_ref):
    x = x_ref[...].astype(jnp.bfloat16)
    u = jnp.dot(x, wh_ref[...], preferred_element_type=jnp.float32)
    lane = jax.lax.broadcasted_iota(jnp.int32, (1, HPAD), 1)
    tgt = tgt_ref[0, 0, :]
    hidx = jnp.clip(tgt, 0, C0 - 1)
    s = jnp.sum(jnp.exp(u), axis=1) - (HPAD - HDIM)
    tval = jnp.sum(jnp.where(lane == hidx[:, None], u, 0.0), axis=1)
    hq_ref[0, 0, :] = jnp.log(s)
    hq_ref[0, 1, :] = tval
    hq_ref[0, 2, :] = u[:, C0]
    hq_ref[0, 3, :] = u[:, C0 + 1]
    hq_ref[0, 4, :] = u[:, C0 + 2]
    zero = jnp.zeros((B,), jnp.float32)
    hq_ref[0, 5, :] = zero
    hq_ref[0, 6, :] = zero
    hq_ref[0, 7, :] = zero

    t0 = jnp.dot(x, p0_ref[...], preferred_element_type=jnp.float32)
    t1 = jnp.dot(x, p1_ref[...], preferred_element_type=jnp.float32)
    t2 = jnp.dot(x, p2_ref[...], preferred_element_type=jnp.float32)
    t_ref[:, TOFF[0]:TOFF[0] + TD[0]] = t0.astype(jnp.bfloat16)
    t_ref[:, TOFF[1]:TOFF[1] + TD[1]] = t1.astype(jnp.bfloat16)
    t_ref[:, TOFF[2]:TOFF[2] + TD[2]] = t2.astype(jnp.bfloat16)


def _k2_body(tgt_ref, hq_ref, t_ref, s0_ref, s1_ref, s2_ref,
             loss_ref, accs_ref, acct_ref):
    i = pl.program_id(0)
    k = pl.program_id(1)
    lane = jax.lax.broadcasted_iota(jnp.int32, (1, KC), 1)

    def tail(j, s_chunk_ref):
        cnt = CUT[j + 1] - CUT[j]

        @pl.when(k < _nchunks(j))
        def _():
            sc = s_chunk_ref[...]
            base = k * KC

            def body(b, carry):
                tb = t_ref[pl.ds(b * B, B), TOFF[j]:TOFF[j] + TD[j]]
                uu = jnp.dot(tb, sc, preferred_element_type=jnp.float32)
                tgt = tgt_ref[b, :]
                relk = (jnp.clip(tgt - CUT[j], 0, cnt - 1) - base)[:, None]
                sp = _fold128(jnp.exp(uu))
                tv = _fold128(jnp.where(lane == relk, uu, 0.0))
                row = pl.ds(b * B, B)
                prev_s = jnp.where(k == 0, 0.0, accs_ref[j, row, :])
                prev_t = jnp.where(k == 0, 0.0, acct_ref[j, row, :])
                accs_ref[j, row, :] = prev_s + sp
                acct_ref[j, row, :] = prev_t + tv
                return carry

            jax.lax.fori_loop(0, N // B, body, 0)

    @pl.when(i == 0)
    def _():
        tail(0, s0_ref)

    @pl.when(i == 1)
    def _():
        tail(1, s1_ref)

    @pl.when(i == 2)
    def _():
        tail(2, s2_ref)

    @pl.when((i == 2) & (k == _nchunks(2) - 1))
    def _():
        def cbody(b, acc_vec):
            tgt = tgt_ref[b, :]
            row = pl.ds(b * B, B)
            ls_head = hq_ref[b, 0, :]
            lp = hq_ref[b, 1, :] - ls_head
            for j in range(NTAIL):
                lo, hi = CUT[j], CUT[j + 1]
                s_j = jnp.sum(accs_ref[j, row, :], axis=1) - _npad(j)
                tv_j = jnp.sum(acct_ref[j, row, :], axis=1)
                lp_j = (hq_ref[b, 2 + j, :] - ls_head + tv_j - jnp.log(s_j))
                lp = jnp.where((tgt >= lo) & (tgt < hi), lp_j, lp)
            return acc_vec + jnp.sum(lp.reshape(B // 128, 128), axis=0)

        acc = jax.lax.fori_loop(0, N // B, cbody,
                                jnp.zeros((128,), jnp.float32))
        total = jnp.sum(acc) * (-1.0 / N)
        loss_ref[0, :] = jnp.full((128,), total)


def kernel(logits, targets, W_head, P0, S0, P1, S1, P2, S2):
    whp = jnp.pad(W_head, ((0, 0), (0, HPAD - HDIM))).astype(jnp.bfloat16)
    p0b = P0.astype(jnp.bfloat16)
    p1b = P1.astype(jnp.bfloat16)
    p2b = P2.astype(jnp.bfloat16)
    s0b = jnp.pad(S0, ((0, 0), (0, _npad(0)))).astype(jnp.bfloat16)
    s1b = jnp.pad(S1, ((0, 0), (0, _npad(1)))).astype(jnp.bfloat16)
    s2b = jnp.pad(S2, ((0, 0), (0, _npad(2)))).astype(jnp.bfloat16)
    tgt2 = targets.astype(jnp.int32).reshape(N // B, B)
    tgt3 = tgt2.reshape(N // B, 1, B)

    t_buf, hq = pl.pallas_call(
        _k1_body,
        grid=(N // B,),
        in_specs=[
            pl.BlockSpec((B, C), lambda b: (b, 0)),
            pl.BlockSpec((C, HPAD), lambda b: (0, 0)),
            pl.BlockSpec((C, TD[0]), lambda b: (0, 0)),
            pl.BlockSpec((C, TD[1]), lambda b: (0, 0)),
            pl.BlockSpec((C, TD[2]), lambda b: (0, 0)),
            pl.BlockSpec((1, 1, B), lambda b: (b, 0, 0)),
        ],
        out_specs=[
            pl.BlockSpec((B, TW), lambda b: (b, 0)),
            pl.BlockSpec((1, 8, B), lambda b: (b, 0, 0)),
        ],
        out_shape=[
            jax.ShapeDtypeStruct((N, TW), jnp.bfloat16),
            jax.ShapeDtypeStruct((N // B, 8, B), jnp.float32),
        ],
    )(logits, whp, p0b, p1b, p2b, tgt3)

    nk = _nchunks(2)
    loss = pl.pallas_call(
        _k2_body,
        grid=(NTAIL, nk),
        in_specs=[
            pl.BlockSpec((N // B, B), lambda i, k: (0, 0)),
            pl.BlockSpec((N // B, 8, B), lambda i, k: (0, 0, 0)),
            pl.BlockSpec((N, TW), lambda i, k: (0, 0)),
            pl.BlockSpec((TD[0], KC),
                         lambda i, k: (0, jnp.where(i == 0, jnp.minimum(k, _nchunks(0) - 1), 0))),
            pl.BlockSpec((TD[1], KC),
                         lambda i, k: (0, jnp.where(i == 1, k, 0))),
            pl.BlockSpec((TD[2], KC),
                         lambda i, k: (0, jnp.where(i == 2, k, 0))),
        ],
        out_specs=pl.BlockSpec((1, 128), lambda i, k: (0, 0)),
        out_shape=jax.ShapeDtypeStruct((1, 128), jnp.float32),
        scratch_shapes=[
            pltpu.VMEM((NTAIL, N, 128), jnp.float32),
            pltpu.VMEM((NTAIL, N, 128), jnp.float32),
        ],
    )(tgt2, hq, t_buf, s0b, s1b, s2b)

    return loss[0, 0]
